# baseline (speedup 1.0000x reference)
import jax
import jax.numpy as jnp
import numpy as np
from jax import lax
from jax.experimental import pallas as pl
from jax.experimental.pallas import tpu as pltpu

B, C, H, W = 8, 192, 24, 24
E, TOPK = 8, 2
HW = H * W
KC = 9 * C

_PERM = np.zeros((KC, KC), dtype=np.float32)
_c = np.repeat(np.arange(C), 9)
_k = np.tile(np.arange(9), C)
_PERM[_c * 9 + _k, _k * C + _c] = 1.0


def _prep_kernel(a_ref, b_ref, wpre_ref, bpre_ref,
                 fc0w_ref, fc0b_ref, fc1w_ref, fc1b_ref,
                 wnat_ref, perm_ref,
                 x9_ref, idx_ref, cof_ref, w9_ref):
    w9_ref[0] = jnp.dot(wnat_ref[0].astype(jnp.bfloat16),
                        perm_ref[...].astype(jnp.bfloat16),
                        preferred_element_type=jnp.float32
                        ).astype(jnp.bfloat16)
    af = a_ref[0].reshape(C, HW)
    bf = b_ref[0].reshape(C, HW)
    wpa = wpre_ref[:, :C]
    wpb = wpre_ref[:, C:]
    x = jnp.dot(wpa, af, preferred_element_type=jnp.float32)
    x = x + jnp.dot(wpb, bf, preferred_element_type=jnp.float32)
    x = x + bpre_ref[...]
    x = jnp.where(x >= 0, x, 0.01 * x)

    w_of_p = lax.broadcasted_iota(jnp.int32, (C, HW), 1) % W
    xh = x.astype(jnp.bfloat16)
    for dyi in range(3):
        for dxi in range(3):
            s = (dyi - 1) * W + (dxi - 1)
            if s > 0:
                sh = jnp.concatenate(
                    [xh[:, s:], jnp.zeros((C, s), jnp.bfloat16)], axis=1)
            elif s < 0:
                sh = jnp.concatenate(
                    [jnp.zeros((C, -s), jnp.bfloat16), xh[:, :s]], axis=1)
            else:
                sh = xh
            if dxi == 0:
                sh = jnp.where(w_of_p == 0, jnp.bfloat16(0), sh)
            elif dxi == 2:
                sh = jnp.where(w_of_p == W - 1, jnp.bfloat16(0), sh)
            base = (dyi * 3 + dxi) * C
            x9_ref[0, base:base + C, :] = sh

    pooled = (jnp.max(x, axis=1, keepdims=True)
              + jnp.mean(x, axis=1, keepdims=True))
    g = jnp.dot(fc1w_ref[...], pooled,
                preferred_element_type=jnp.float32) + fc1b_ref[...]
    g = jnp.where(g >= 0, g, 0.2 * g)
    z = jnp.dot(fc0w_ref[...], pooled,
                preferred_element_type=jnp.float32) + fc0b_ref[...]
    noise = jnp.maximum(z, 0.0) + jnp.log1p(jnp.exp(-jnp.abs(z)))
    nmean = jnp.mean(noise)
    nstd = jnp.sqrt(jnp.sum((noise - nmean) ** 2) / (E - 1))
    nstd = jnp.where(nstd == 0, 1.0, nstd)
    t = g + (noise - nmean) / nstd

    iota = lax.broadcasted_iota(jnp.int32, (E, 1), 0)
    m1 = jnp.max(t)
    i1 = jnp.min(jnp.where(t == m1, iota, E))
    t2 = jnp.where(iota == i1, -jnp.float32(1e30), t)
    m2 = jnp.max(t2)
    i2 = jnp.min(jnp.where(t2 == m2, iota, E))
    g1 = jnp.sum(jnp.where(iota == i1, g, 0.0))
    g2 = jnp.sum(jnp.where(iota == i2, g, 0.0))
    mm = jnp.maximum(g1, g2)
    e1 = jnp.exp(g1 - mm)
    e2 = jnp.exp(g2 - mm)
    zs = e1 + e2
    lane2 = lax.broadcasted_iota(jnp.int32, (1, 1, TOPK), 2)
    idx_ref[...] = jnp.where(lane2 == 0, i1, i2)
    cof_ref[...] = jnp.where(lane2 == 0, e1 / zs, e2 / zs)


def _expert_kernel(idx_sref, cof_sref, x9_ref, w_ref, bias_ref, out_ref):
    i = pl.program_id(0)
    y = jnp.dot(w_ref[0], x9_ref[0], preferred_element_type=jnp.float32)
    y = y + bias_ref[0]
    y = cof_sref[i] * y

    @pl.when(i % 2 == 0)
    def _():
        out_ref[0] = y

    @pl.when(i % 2 == 1)
    def _():
        out_ref[0] += y


def kernel(a, b, w_pre, b_pre, fc0_w, fc0_b, fc1_w, fc1_b,
           expert_w, expert_b):
    bpre = b_pre.reshape(C, 1)
    fc0b = fc0_b.reshape(E, 1)
    fc1b = fc1_b.reshape(E, 1)

    wnat = expert_w.reshape(E, C, KC)
    perm = jnp.asarray(_PERM)

    x9, idx, cof, wfull = pl.pallas_call(
        _prep_kernel,
        grid=(B,),
        in_specs=[
            pl.BlockSpec((1, C, H, W), lambda i: (i, 0, 0, 0)),
            pl.BlockSpec((1, C, H, W), lambda i: (i, 0, 0, 0)),
            pl.BlockSpec((C, 2 * C), lambda i: (0, 0)),
            pl.BlockSpec((C, 1), lambda i: (0, 0)),
            pl.BlockSpec((E, C), lambda i: (0, 0)),
            pl.BlockSpec((E, 1), lambda i: (0, 0)),
            pl.BlockSpec((E, C), lambda i: (0, 0)),
            pl.BlockSpec((E, 1), lambda i: (0, 0)),
            pl.BlockSpec((1, C, KC), lambda i: (i, 0, 0)),
            pl.BlockSpec((KC, KC), lambda i: (0, 0)),
        ],
        out_specs=[
            pl.BlockSpec((1, KC, HW), lambda i: (i, 0, 0)),
            pl.BlockSpec((1, 1, TOPK), lambda i: (i, 0, 0)),
            pl.BlockSpec((1, 1, TOPK), lambda i: (i, 0, 0)),
            pl.BlockSpec((1, C, KC), lambda i: (i, 0, 0)),
        ],
        out_shape=[
            jax.ShapeDtypeStruct((B, KC, HW), jnp.bfloat16),
            jax.ShapeDtypeStruct((B, 1, TOPK), jnp.int32),
            jax.ShapeDtypeStruct((B, 1, TOPK), jnp.float32),
            jax.ShapeDtypeStruct((E, C, KC), jnp.bfloat16),
        ],
    )(a, b, w_pre, bpre, fc0_w, fc0b, fc1_w, fc1b, wnat, perm)

    idx_flat = idx.reshape(B * TOPK)
    cof_flat = cof.reshape(B * TOPK)
    biasr = expert_b.reshape(E, C, 1)

    grid_spec = pltpu.PrefetchScalarGridSpec(
        num_scalar_prefetch=2,
        grid=(B * TOPK,),
        in_specs=[
            pl.BlockSpec((1, KC, HW), lambda i, idx_s, cof_s: (i // 2, 0, 0)),
            pl.BlockSpec((1, C, KC), lambda i, idx_s, cof_s: (idx_s[i], 0, 0)),
            pl.BlockSpec((1, C, 1), lambda i, idx_s, cof_s: (idx_s[i], 0, 0)),
        ],
        out_specs=pl.BlockSpec((1, C, HW),
                               lambda i, idx_s, cof_s: (i // 2, 0, 0)),
    )
    out = pl.pallas_call(
        _expert_kernel,
        grid_spec=grid_spec,
        out_shape=jax.ShapeDtypeStruct((B, C, HW), jnp.float32),
    )(idx_flat, cof_flat, x9, wfull, biasr)

    return out.reshape(B, C, H, W)


# speedup vs baseline: 3.1199x; 3.1199x over previous
import jax
import jax.numpy as jnp
from jax import lax
from jax.experimental import pallas as pl
from jax.experimental.pallas import tpu as pltpu

B, C, H, W = 8, 192, 24, 24
E, TOPK = 8, 2
HW = H * W


def _prep_kernel(a_ref, b_ref, wpre_ref, bpre_ref,
                 fc0w_ref, fc0b_ref, fc1w_ref, fc1b_ref,
                 xt_ref, idx_ref, cof_ref):
    wpa = wpre_ref[:, :C]
    wpb = wpre_ref[:, C:]
    dn = (((1,), (1,)), ((), ()))
    x = lax.dot_general(a_ref[0], wpa, dn,
                        preferred_element_type=jnp.float32)
    x = x + lax.dot_general(b_ref[0], wpb, dn,
                            preferred_element_type=jnp.float32)
    x = x + bpre_ref[...]
    x = jnp.where(x >= 0, x, 0.01 * x)
    xt_ref[0] = x.astype(jnp.bfloat16)

    pooled = (jnp.max(x, axis=0, keepdims=True)
              + jnp.mean(x, axis=0, keepdims=True))
    g = lax.dot_general(pooled, fc1w_ref[...], dn,
                        preferred_element_type=jnp.float32) + fc1b_ref[...]
    g = jnp.where(g >= 0, g, 0.2 * g)
    z = lax.dot_general(pooled, fc0w_ref[...], dn,
                        preferred_element_type=jnp.float32) + fc0b_ref[...]
    noise = jnp.maximum(z, 0.0) + jnp.log1p(jnp.exp(-jnp.abs(z)))
    nmean = jnp.mean(noise)
    nstd = jnp.sqrt(jnp.sum((noise - nmean) ** 2) / (E - 1))
    nstd = jnp.where(nstd == 0, 1.0, nstd)
    t = g + (noise - nmean) / nstd

    iota = lax.broadcasted_iota(jnp.int32, (1, E), 1)
    m1 = jnp.max(t)
    i1 = jnp.min(jnp.where(t == m1, iota, E))
    t2 = jnp.where(iota == i1, -jnp.float32(1e30), t)
    m2 = jnp.max(t2)
    i2 = jnp.min(jnp.where(t2 == m2, iota, E))
    g1 = jnp.sum(jnp.where(iota == i1, g, 0.0))
    g2 = jnp.sum(jnp.where(iota == i2, g, 0.0))
    mm = jnp.maximum(g1, g2)
    e1 = jnp.exp(g1 - mm)
    e2 = jnp.exp(g2 - mm)
    zs = e1 + e2
    lane2 = lax.broadcasted_iota(jnp.int32, (1, 1, TOPK), 2)
    idx_ref[...] = jnp.where(lane2 == 0, i1, i2)
    cof_ref[...] = jnp.where(lane2 == 0, e1 / zs, e2 / zs)


def _expert_kernel(idx_sref, cof_sref, xt_ref, w_ref, bias_ref, out_ref):
    i = pl.program_id(0)
    x = xt_ref[0]
    p_in_row = lax.broadcasted_iota(jnp.int32, (HW, C), 0) % W
    dn = (((1,), (1,)), ((), ()))
    acc = jnp.zeros((HW, C), jnp.float32)
    for ky in range(3):
        for kx in range(3):
            s = (ky - 1) * W + (kx - 1)
            if s > 0:
                sh = jnp.concatenate(
                    [x[s:], jnp.zeros((s, C), jnp.bfloat16)], axis=0)
            elif s < 0:
                sh = jnp.concatenate(
                    [jnp.zeros((-s, C), jnp.bfloat16), x[:s]], axis=0)
            else:
                sh = x
            if kx == 0:
                sh = jnp.where(p_in_row == 0, jnp.bfloat16(0), sh)
            elif kx == 2:
                sh = jnp.where(p_in_row == W - 1, jnp.bfloat16(0), sh)
            wk = w_ref[0, ky * 3 + kx].astype(jnp.bfloat16)
            acc = acc + lax.dot_general(sh, wk, dn,
                                        preferred_element_type=jnp.float32)
    y = acc + bias_ref[0]
    y = cof_sref[i // 2, 0, i % 2] * y

    @pl.when(i % 2 == 0)
    def _():
        out_ref[0] = y

    @pl.when(i % 2 == 1)
    def _():
        out_ref[0] += y


def kernel(a, b, w_pre, b_pre, fc0_w, fc0_b, fc1_w, fc1_b,
           expert_w, expert_b):
    at = a.transpose(0, 2, 3, 1).reshape(B, HW, C)
    bt = b.transpose(0, 2, 3, 1).reshape(B, HW, C)
    wt = expert_w.transpose(0, 3, 4, 1, 2).reshape(E, 9, C, C)
    bpre = b_pre.reshape(1, C)
    fc0b = fc0_b.reshape(1, E)
    fc1b = fc1_b.reshape(1, E)
    biasr = expert_b.reshape(E, 1, C)

    xt, idx, cof = pl.pallas_call(
        _prep_kernel,
        grid=(B,),
        in_specs=[
            pl.BlockSpec((1, HW, C), lambda i: (i, 0, 0)),
            pl.BlockSpec((1, HW, C), lambda i: (i, 0, 0)),
            pl.BlockSpec((C, 2 * C), lambda i: (0, 0)),
            pl.BlockSpec((1, C), lambda i: (0, 0)),
            pl.BlockSpec((E, C), lambda i: (0, 0)),
            pl.BlockSpec((1, E), lambda i: (0, 0)),
            pl.BlockSpec((E, C), lambda i: (0, 0)),
            pl.BlockSpec((1, E), lambda i: (0, 0)),
        ],
        out_specs=[
            pl.BlockSpec((1, HW, C), lambda i: (i, 0, 0)),
            pl.BlockSpec((1, 1, TOPK), lambda i: (i, 0, 0)),
            pl.BlockSpec((1, 1, TOPK), lambda i: (i, 0, 0)),
        ],
        out_shape=[
            jax.ShapeDtypeStruct((B, HW, C), jnp.bfloat16),
            jax.ShapeDtypeStruct((B, 1, TOPK), jnp.int32),
            jax.ShapeDtypeStruct((B, 1, TOPK), jnp.float32),
        ],
    )(at, bt, w_pre, bpre, fc0_w, fc0b, fc1_w, fc1b)

    grid_spec = pltpu.PrefetchScalarGridSpec(
        num_scalar_prefetch=2,
        grid=(B * TOPK,),
        in_specs=[
            pl.BlockSpec((1, HW, C), lambda i, idx_s, cof_s: (i // 2, 0, 0)),
            pl.BlockSpec((1, 9, C, C),
                         lambda i, idx_s, cof_s: (idx_s[i // 2, 0, i % 2],
                                                  0, 0, 0)),
            pl.BlockSpec((1, 1, C),
                         lambda i, idx_s, cof_s: (idx_s[i // 2, 0, i % 2],
                                                  0, 0)),
        ],
        out_specs=pl.BlockSpec((1, HW, C),
                               lambda i, idx_s, cof_s: (i // 2, 0, 0)),
    )
    out_t = pl.pallas_call(
        _expert_kernel,
        grid_spec=grid_spec,
        out_shape=jax.ShapeDtypeStruct((B, HW, C), jnp.float32),
    )(idx, cof, xt, wt, biasr)

    return out_t.reshape(B, H, W, C).transpose(0, 3, 1, 2)


# speedup vs baseline: 3.8742x; 1.2418x over previous
import jax
import jax.numpy as jnp
from jax import lax
from jax.experimental import pallas as pl
from jax.experimental.pallas import tpu as pltpu

B, C, H, W = 8, 192, 24, 24
E, TOPK = 8, 2
HW = H * W


def _prep_kernel(a_ref, b_ref, wpre_ref, bpre_ref,
                 fc0w_ref, fc0b_ref, fc1w_ref, fc1b_ref,
                 xt_ref, idx_ref, cof_ref):
    wpa = wpre_ref[:, :C]
    wpb = wpre_ref[:, C:]
    dn = (((1,), (1,)), ((), ()))
    x = lax.dot_general(a_ref[0], wpa, dn,
                        preferred_element_type=jnp.float32)
    x = x + lax.dot_general(b_ref[0], wpb, dn,
                            preferred_element_type=jnp.float32)
    x = x + bpre_ref[...]
    x = jnp.where(x >= 0, x, 0.01 * x)
    xt_ref[0] = x.astype(jnp.bfloat16)

    pooled = (jnp.max(x, axis=0, keepdims=True)
              + jnp.mean(x, axis=0, keepdims=True))
    g = lax.dot_general(pooled, fc1w_ref[...], dn,
                        preferred_element_type=jnp.float32) + fc1b_ref[...]
    g = jnp.where(g >= 0, g, 0.2 * g)
    z = lax.dot_general(pooled, fc0w_ref[...], dn,
                        preferred_element_type=jnp.float32) + fc0b_ref[...]
    noise = jnp.maximum(z, 0.0) + jnp.log1p(jnp.exp(-jnp.abs(z)))
    nmean = jnp.mean(noise)
    nstd = jnp.sqrt(jnp.sum((noise - nmean) ** 2) / (E - 1))
    nstd = jnp.where(nstd == 0, 1.0, nstd)
    t = g + (noise - nmean) / nstd

    iota = lax.broadcasted_iota(jnp.int32, (1, E), 1)
    m1 = jnp.max(t)
    i1 = jnp.min(jnp.where(t == m1, iota, E))
    t2 = jnp.where(iota == i1, -jnp.float32(1e30), t)
    m2 = jnp.max(t2)
    i2 = jnp.min(jnp.where(t2 == m2, iota, E))
    g1 = jnp.sum(jnp.where(iota == i1, g, 0.0))
    g2 = jnp.sum(jnp.where(iota == i2, g, 0.0))
    mm = jnp.maximum(g1, g2)
    e1 = jnp.exp(g1 - mm)
    e2 = jnp.exp(g2 - mm)
    zs = e1 + e2
    lane2 = lax.broadcasted_iota(jnp.int32, (1, 1, TOPK), 2)
    idx_ref[...] = jnp.where(lane2 == 0, i1, i2)
    cof_ref[...] = jnp.where(lane2 == 0, e1 / zs, e2 / zs)


def _expert_kernel(idx_sref, cof_sref, xt_ref, w0_ref, w1_ref, bias_ref,
                   out_ref):
    i = pl.program_id(0)
    x = xt_ref[0]
    p_in_row = lax.broadcasted_iota(jnp.int32, (HW, C), 0) % W
    dn = (((1,), (1,)), ((), ()))

    shifts = []
    for ky in range(3):
        for kx in range(3):
            s = (ky - 1) * W + (kx - 1)
            if s > 0:
                sh = jnp.concatenate(
                    [x[s:], jnp.zeros((s, C), jnp.bfloat16)], axis=0)
            elif s < 0:
                sh = jnp.concatenate(
                    [jnp.zeros((-s, C), jnp.bfloat16), x[:s]], axis=0)
            else:
                sh = x
            if kx == 0:
                sh = jnp.where(p_in_row == 0, jnp.bfloat16(0), sh)
            elif kx == 2:
                sh = jnp.where(p_in_row == W - 1, jnp.bfloat16(0), sh)
            shifts.append(sh)

    y = jnp.zeros((HW, C), jnp.float32)
    for slot, w_ref in ((0, w0_ref), (1, w1_ref)):
        acc = jnp.zeros((HW, C), jnp.float32)
        for k in range(9):
            wk = w_ref[0, k].astype(jnp.bfloat16)
            acc = acc + lax.dot_general(shifts[k], wk, dn,
                                        preferred_element_type=jnp.float32)
        e = idx_sref[i, 0, slot]
        acc = acc + bias_ref[pl.ds(e, 1), :]
        y = y + cof_sref[i, 0, slot] * acc
    out_ref[0] = y


def kernel(a, b, w_pre, b_pre, fc0_w, fc0_b, fc1_w, fc1_b,
           expert_w, expert_b):
    at = a.transpose(0, 2, 3, 1).reshape(B, HW, C)
    bt = b.transpose(0, 2, 3, 1).reshape(B, HW, C)
    wt = expert_w.transpose(0, 3, 4, 1, 2).reshape(E, 9, C, C)
    bpre = b_pre.reshape(1, C)
    fc0b = fc0_b.reshape(1, E)
    fc1b = fc1_b.reshape(1, E)

    xt, idx, cof = pl.pallas_call(
        _prep_kernel,
        grid=(B,),
        in_specs=[
            pl.BlockSpec((1, HW, C), lambda i: (i, 0, 0)),
            pl.BlockSpec((1, HW, C), lambda i: (i, 0, 0)),
            pl.BlockSpec((C, 2 * C), lambda i: (0, 0)),
            pl.BlockSpec((1, C), lambda i: (0, 0)),
            pl.BlockSpec((E, C), lambda i: (0, 0)),
            pl.BlockSpec((1, E), lambda i: (0, 0)),
            pl.BlockSpec((E, C), lambda i: (0, 0)),
            pl.BlockSpec((1, E), lambda i: (0, 0)),
        ],
        out_specs=[
            pl.BlockSpec((1, HW, C), lambda i: (i, 0, 0)),
            pl.BlockSpec((1, 1, TOPK), lambda i: (i, 0, 0)),
            pl.BlockSpec((1, 1, TOPK), lambda i: (i, 0, 0)),
        ],
        out_shape=[
            jax.ShapeDtypeStruct((B, HW, C), jnp.bfloat16),
            jax.ShapeDtypeStruct((B, 1, TOPK), jnp.int32),
            jax.ShapeDtypeStruct((B, 1, TOPK), jnp.float32),
        ],
    )(at, bt, w_pre, bpre, fc0_w, fc0b, fc1_w, fc1b)

    grid_spec = pltpu.PrefetchScalarGridSpec(
        num_scalar_prefetch=2,
        grid=(B,),
        in_specs=[
            pl.BlockSpec((1, HW, C), lambda i, idx_s, cof_s: (i, 0, 0)),
            pl.BlockSpec((1, 9, C, C),
                         lambda i, idx_s, cof_s: (idx_s[i, 0, 0], 0, 0, 0)),
            pl.BlockSpec((1, 9, C, C),
                         lambda i, idx_s, cof_s: (idx_s[i, 0, 1], 0, 0, 0)),
            pl.BlockSpec((E, C), lambda i, idx_s, cof_s: (0, 0)),
        ],
        out_specs=pl.BlockSpec((1, HW, C),
                               lambda i, idx_s, cof_s: (i, 0, 0)),
    )
    out_t = pl.pallas_call(
        _expert_kernel,
        grid_spec=grid_spec,
        out_shape=jax.ShapeDtypeStruct((B, HW, C), jnp.float32),
    )(idx, cof, xt, wt, wt, expert_b)

    return out_t.reshape(B, H, W, C).transpose(0, 3, 1, 2)
